# baseline (speedup 1.0000x reference)
import jax
import jax.numpy as jnp
from jax.experimental import pallas as pl
from jax.experimental.pallas import tpu as pltpu

B = 2048
IN = 1024
OUT = 1024
E = 8

BC = 512
NB = B // BC
CH = 512
NC = OUT // CH


def _moe_body(x_ref, wg_ref, bg_ref, be_ref, we_ref, out_ref, s_ref, c_ref):
    i = pl.program_id(0)

    @pl.when(i < NB)
    def _gate():
        xx = x_ref[...]
        logits = jax.lax.dot_general(
            wg_ref[...], xx, (((1,), (1,)), ((), ())),
            preferred_element_type=jnp.float32) + bg_ref[...]
        iota = jax.lax.broadcasted_iota(jnp.int32, (E, BC), 0)
        v1 = jnp.max(logits, axis=0, keepdims=True)
        i1 = jnp.min(jnp.where(logits == v1, iota, E + 1), axis=0,
                     keepdims=True)
        masked = jnp.where(iota == i1, -jnp.inf, logits)
        v2 = jnp.max(masked, axis=0, keepdims=True)
        i2 = jnp.min(jnp.where(masked == v2, iota, E + 1), axis=0,
                     keepdims=True)
        t = jnp.exp(v2 - v1)
        w1 = 1.0 / (1.0 + t)
        w2 = t / (1.0 + t)
        g = jnp.where(iota == i1, w1, 0.0) + jnp.where(iota == i2, w2, 0.0)
        sc = jax.lax.dot_general(
            g, xx, (((1,), (0,)), ((), ())),
            preferred_element_type=jnp.float32)
        cc = jnp.sum(g, axis=1, keepdims=True)

        @pl.when(i == 0)
        def _init():
            s_ref[...] = sc
            c_ref[...] = cc

        @pl.when(i > 0)
        def _accum():
            s_ref[...] += sc
            c_ref[...] += cc

    @pl.when(i >= NB)
    def _expert():
        acc = jax.lax.dot_general(
            c_ref[...], be_ref[...], (((0,), (0,)), ((), ())),
            preferred_element_type=jnp.float32)
        for e in range(E):
            acc += jax.lax.dot_general(
                s_ref[e:e + 1, :], we_ref[e], (((1,), (1,)), ((), ())),
                preferred_element_type=jnp.float32)
        out_ref[...] = jnp.broadcast_to(acc, (B, CH))


def kernel(x, Wg, bg, We, be):
    bg2 = bg.reshape(E, 1)
    return pl.pallas_call(
        _moe_body,
        grid=(NB + NC,),
        in_specs=[
            pl.BlockSpec((BC, IN), lambda i: (jnp.minimum(i, NB - 1), 0)),
            pl.BlockSpec((E, IN), lambda i: (0, 0)),
            pl.BlockSpec((E, 1), lambda i: (0, 0)),
            pl.BlockSpec((E, CH), lambda i: (0, jnp.maximum(i - NB, 0))),
            pl.BlockSpec((E, CH, IN), lambda i: (0, jnp.maximum(i - NB, 0), 0)),
        ],
        out_specs=pl.BlockSpec((B, CH), lambda i: (0, jnp.maximum(i - NB, 0))),
        out_shape=jax.ShapeDtypeStruct((B, OUT), jnp.float32),
        scratch_shapes=[
            pltpu.VMEM((E, IN), jnp.float32),
            pltpu.VMEM((E, 1), jnp.float32),
        ],
    )(x, Wg, bg2, be, We)


# speedup vs baseline: 1.0863x; 1.0863x over previous
import jax
import jax.numpy as jnp
from jax.experimental import pallas as pl
from jax.experimental.pallas import tpu as pltpu

B = 2048
IN = 1024
OUT = 1024
E = 8

BC = 1024
NB = B // BC
CH = 256
NC = OUT // CH


def _moe_body(x_ref, wg_ref, bg_ref, be_ref, we_ref, out_ref, s_ref, c_ref):
    i = pl.program_id(0)

    @pl.when(i < NB)
    def _gate():
        xx = x_ref[...]
        logits = jax.lax.dot_general(
            wg_ref[...], xx, (((1,), (1,)), ((), ())),
            preferred_element_type=jnp.float32) + bg_ref[...]
        iota = jax.lax.broadcasted_iota(jnp.int32, (E, BC), 0)
        v1 = jnp.max(logits, axis=0, keepdims=True)
        i1 = jnp.min(jnp.where(logits == v1, iota, E + 1), axis=0,
                     keepdims=True)
        masked = jnp.where(iota == i1, -jnp.inf, logits)
        v2 = jnp.max(masked, axis=0, keepdims=True)
        i2 = jnp.min(jnp.where(masked == v2, iota, E + 1), axis=0,
                     keepdims=True)
        t = jnp.exp(v2 - v1)
        w1 = 1.0 / (1.0 + t)
        w2 = t / (1.0 + t)
        g = jnp.where(iota == i1, w1, 0.0) + jnp.where(iota == i2, w2, 0.0)
        sc = jax.lax.dot_general(
            g, xx, (((1,), (0,)), ((), ())),
            preferred_element_type=jnp.float32)
        cc = jnp.sum(g, axis=1, keepdims=True)

        @pl.when(i == 0)
        def _init():
            s_ref[...] = sc
            c_ref[...] = cc

        @pl.when(i > 0)
        def _accum():
            s_ref[...] += sc
            c_ref[...] += cc

    @pl.when(i >= NB)
    def _expert():
        acc = jax.lax.dot_general(
            c_ref[...], be_ref[...], (((0,), (0,)), ((), ())),
            preferred_element_type=jnp.float32)
        for e in range(E):
            acc += jax.lax.dot_general(
                s_ref[e:e + 1, :], we_ref[e], (((1,), (1,)), ((), ())),
                preferred_element_type=jnp.float32)
        out_ref[...] = jnp.broadcast_to(acc, (B, CH))


def kernel(x, Wg, bg, We, be):
    bg2 = bg.reshape(E, 1)
    return pl.pallas_call(
        _moe_body,
        grid=(NB + NC,),
        in_specs=[
            pl.BlockSpec((BC, IN), lambda i: (jnp.minimum(i, NB - 1), 0)),
            pl.BlockSpec((E, IN), lambda i: (0, 0)),
            pl.BlockSpec((E, 1), lambda i: (0, 0)),
            pl.BlockSpec((E, CH), lambda i: (0, jnp.maximum(i - NB, 0))),
            pl.BlockSpec((E, CH, IN), lambda i: (0, jnp.maximum(i - NB, 0), 0)),
        ],
        out_specs=pl.BlockSpec((B, CH), lambda i: (0, jnp.maximum(i - NB, 0))),
        out_shape=jax.ShapeDtypeStruct((B, OUT), jnp.float32),
        scratch_shapes=[
            pltpu.VMEM((E, IN), jnp.float32),
            pltpu.VMEM((E, 1), jnp.float32),
        ],
    )(x, Wg, bg2, be, We)
